# baseline (speedup 1.0000x reference)
import functools

import jax
import jax.numpy as jnp
from jax import lax
from jax.experimental import pallas as pl
from jax.experimental.pallas import tpu as pltpu
from jax.experimental.pallas import tpu_sc as plsc

N_IMG, C_CH, H_F, W_F = 4, 384, 64, 64
R_ROIS = 1024
P_OUT = 49
NC, NS, L = 2, 16, 16
NW = NC * NS
RW = R_ROIS // NW
PW = RW * P_OUT
N_CHUNK = PW // L
SLAB_D = 3
SLAB_SZ = C_CH * P_OUT


def _pre_body(f_ref, t_ref):
    hw = H_F * W_F
    x = f_ref[0].reshape(C_CH, hw)
    x = x.reshape(2, C_CH // 2, hw).transpose(1, 0, 2)
    t_ref[...] = x.reshape(C_CH, hw).astype(jnp.bfloat16).T


def _features_to_table(features):
    return pl.pallas_call(
        _pre_body,
        grid=(N_IMG,),
        in_specs=[pl.BlockSpec((1, C_CH, H_F, W_F), lambda n: (n, 0, 0, 0))],
        out_specs=pl.BlockSpec((H_F * W_F, C_CH), lambda n: (n, 0)),
        out_shape=jax.ShapeDtypeStruct((N_IMG * H_F * W_F, C_CH),
                                       jnp.bfloat16),
    )(features)


def _sc_body(table_hbm, roist_hbm, out_hbm,
             rois_v, pos_v, idx_v, tap_v, slab_v, sem_g0, sem_g1, sem_f):
    sem_g = (sem_g0, sem_g1)
    wid = lax.axis_index("s") * NC + lax.axis_index("c")
    i32 = jnp.int32
    f32 = jnp.float32
    iota = lax.iota(i32, L)

    pltpu.sync_copy(roist_hbm, rois_v)

    def _bvec(i):
        idx5 = (jnp.broadcast_to(i * L, (L,)).astype(i32) + iota) * 5
        return plsc.load_gather(rois_v, [idx5]).astype(i32)

    def _take16(x, idx):
        return lax.gather(
            x, idx[:, None],
            dimension_numbers=lax.GatherDimensionNumbers(
                offset_dims=(), collapsed_slice_dims=(0,),
                start_index_map=(0,)),
            slice_sizes=(1,),
            mode=lax.GatherScatterMode.PROMISE_IN_BOUNDS)

    def _csum(m):
        x = jnp.where(m, 1, 0).astype(i32)
        k = 1
        while k < L:
            shifted = _take16(x, jnp.maximum(iota - k, 0))
            x = x + jnp.where(iota >= k, shifted, 0)
            k *= 2
        return x

    def _count_body(i, t):
        b = _bvec(i)
        return tuple(t[v] + _csum(b == v)[L - 1] for v in range(N_IMG))

    zero4 = tuple(jnp.zeros((), i32) for _ in range(N_IMG))
    tot = lax.fori_loop(0, R_ROIS // L, _count_body, zero4)
    run = lax.fori_loop(0, 2 * wid, _count_body, zero4)
    off = (jnp.zeros((), i32), tot[0], tot[0] + tot[1],
           tot[0] + tot[1] + tot[2])

    for s in range(RW // L):
        b = _bvec(2 * wid + s)
        pos = jnp.zeros((L,), i32)
        new_run = []
        for v in range(N_IMG):
            m = b == v
            csum = _csum(m)
            pos = jnp.where(
                m, jnp.broadcast_to(off[v] + run[v], (L,)) + csum - 1, pos)
            new_run.append(run[v] + csum[L - 1])
        run = tuple(new_run)
        pos_v[pl.ds(s * L, L)] = pos

    def _issue(c, par):
        g0 = PW * wid + L * c
        r0 = g0 // P_OUT
        p0 = g0 - r0 * P_OUT
        pp = jnp.broadcast_to(p0, (L,)).astype(i32) + iota
        wrap = pp >= P_OUT
        r = jnp.broadcast_to(r0, (L,)).astype(i32) + jnp.where(wrap, 1, 0)
        r = jnp.minimum(r, R_ROIS - 1)
        p = pp - jnp.where(wrap, P_OUT, 0)
        r5 = r * 5
        col = lambda j: plsc.load_gather(rois_v, [r5 + j])
        x1, y1, x2, y2 = col(1), col(2), col(3), col(4)
        b = col(0).astype(i32)
        py = (p.astype(f32) * (1.0 / 7.0)).astype(i32)
        px = p - py * 7
        relx = (px.astype(f32) + 0.5) * (1.0 / 7.0)
        rely = (py.astype(f32) + 0.5) * (1.0 / 7.0)
        xs = relx * (x2 - x1) + x1
        ys = rely * (y2 - y1) + y1
        gx = xs * (0.125 * 2.0 / W_F) - 1.0
        gy = ys * (0.125 * 2.0 / H_F) - 1.0
        x = ((gx + 1.0) * W_F - 1.0) * 0.5
        y = ((gy + 1.0) * H_F - 1.0) * 0.5
        xi0 = (x + 1.0).astype(i32) - 1
        yi0 = (y + 1.0).astype(i32) - 1
        dx = x - xi0.astype(f32)
        dy = y - yi0.astype(f32)
        xi1 = xi0 + 1
        yi1 = yi0 + 1
        wx0 = jnp.where((xi0 >= 0) & (xi0 <= W_F - 1), 1.0 - dx, 0.0)
        wx1 = jnp.where(xi1 <= W_F - 1, dx, 0.0)
        wy0 = jnp.where((yi0 >= 0) & (yi0 <= H_F - 1), 1.0 - dy, 0.0)
        wy1 = jnp.where(yi1 <= H_F - 1, dy, 0.0)
        xc0 = jnp.clip(xi0, 0, W_F - 1)
        xc1 = jnp.clip(xi1, 0, W_F - 1)
        yc0 = jnp.clip(yi0, 0, H_F - 1)
        yc1 = jnp.clip(yi1, 0, H_F - 1)
        base = b * (H_F * W_F)
        idx_v[4 * par + 0, :] = base + yc0 * W_F + xc0
        idx_v[4 * par + 1, :] = base + yc1 * W_F + xc0
        idx_v[4 * par + 2, :] = base + yc0 * W_F + xc1
        idx_v[4 * par + 3, :] = base + yc1 * W_F + xc1
        for t in range(4):
            pltpu.async_copy(table_hbm.at[idx_v.at[4 * par + t]],
                             tap_v.at[4 * par + t], sem_g[par])
        return (wx0 * wy0, wx0 * wy1, wx1 * wy0, wx1 * wy1)

    def _wait_taps(par):
        for t in range(4):
            pltpu.make_async_copy(table_hbm.at[idx_v.at[4 * par + t]],
                                  tap_v.at[4 * par + t], sem_g[par]).wait()

    def _compute(c, par, weights):
        wa, wb, wc, wd = weights
        tp0 = L * c

        def _pt_body(p16, carry):
            tp = tp0 + p16
            rl = tp // P_OUT
            psc = tp - rl * P_OUT
            ysc = psc // 7
            xsc = psc - ysc * 7
            slot = rl - (rl // SLAB_D) * SLAB_D
            lane = jnp.broadcast_to(p16, (L,)).astype(i32)
            bwa = _take16(wa, lane)
            bwb = _take16(wb, lane)
            bwc = _take16(wc, lane)
            bwd = _take16(wd, lane)
            cbase = (jnp.broadcast_to(slot * (C_CH // 2), (L,)).astype(i32)
                     + iota)
            pv = jnp.broadcast_to(psc, (L,)).astype(i32)
            pk = lambda w: plsc.pack(w, w, format=plsc.PackFormat.INTERLEAVED)
            pwa, pwb, pwc, pwd = pk(bwa), pk(bwb), pk(bwc), pk(bwd)
            for j in range(C_CH // (2 * L)):
                sl = pl.ds(j * 2 * L, 2 * L)
                o = (pwa * tap_v[4 * par + 0, p16, sl]
                     + pwb * tap_v[4 * par + 1, p16, sl]
                     + pwc * tap_v[4 * par + 2, p16, sl]
                     + pwd * tap_v[4 * par + 3, p16, sl])
                ow = plsc.bitcast(o, jnp.int32)
                plsc.store_scatter(slab_v, [cbase + j * L, pv], ow)
            return carry

        lax.fori_loop(0, L, _pt_body, 0)

        npv = tp0 // P_OUT
        nd = (tp0 + L) // P_OUT
        @pl.when(nd > npv)
        def _flush():
            rl = nd - 1
            slot = rl - (rl // SLAB_D) * SLAB_D
            grp = rl // L
            vec = pos_v[pl.ds(grp * L, L)]
            posr = _take16(vec, jnp.broadcast_to(rl - grp * L, (L,))
                           .astype(i32))[0]
            @pl.when(npv >= 1)
            def _drain():
                pltpu.make_async_copy(
                    out_hbm.at[0], slab_v.at[pl.ds(0, C_CH // 2)],
                    sem_f).wait()
            pltpu.async_copy(
                slab_v.at[pl.ds(slot * (C_CH // 2), C_CH // 2)],
                out_hbm.at[posr], sem_f)

    w0 = _issue(0, 0)

    def _pair(cc, carry):
        w_even = carry
        c0 = 2 * cc
        w_odd = _issue(c0 + 1, 1)
        _wait_taps(0)
        _compute(c0, 0, w_even)
        w_next = _issue(c0 + 2, 0)
        _wait_taps(1)
        _compute(c0 + 1, 1, w_odd)
        return w_next

    lax.fori_loop(0, N_CHUNK // 2, _pair, w0)

    pltpu.make_async_copy(out_hbm.at[0],
                          slab_v.at[pl.ds(0, C_CH // 2)], sem_f).wait()
    _wait_taps(0)


@functools.partial(
    pl.kernel,
    out_type=jax.ShapeDtypeStruct((R_ROIS, C_CH // 2, P_OUT), jnp.int32),
    mesh=plsc.VectorSubcoreMesh(core_axis_name="c", subcore_axis_name="s",
                                num_cores=NC, num_subcores=NS),
    compiler_params=pltpu.CompilerParams(use_tc_tiling_on_sc=False,
                                         needs_layout_passes=False),
    scratch_types=[
        pltpu.VMEM((5 * R_ROIS,), jnp.float32),
        pltpu.VMEM((RW,), jnp.int32),
        pltpu.VMEM((8, L), jnp.int32),
        pltpu.VMEM((8, L, C_CH), jnp.bfloat16),
        pltpu.VMEM((SLAB_D * (C_CH // 2), P_OUT), jnp.int32),
        pltpu.SemaphoreType.DMA,
        pltpu.SemaphoreType.DMA,
        pltpu.SemaphoreType.DMA,
    ],
)
def _sc_sample(table_hbm, roist_hbm, out_hbm, *scratch):
    _sc_body(table_hbm, roist_hbm, out_hbm, *scratch)


def kernel(features, rois):
    table = _features_to_table(features)
    out = _sc_sample(table, rois.reshape(5 * R_ROIS))
    bf = jax.lax.bitcast_convert_type(out, jnp.bfloat16)
    full = bf.transpose(0, 3, 1, 2).reshape(R_ROIS, C_CH, P_OUT)
    return full.astype(jnp.float32).reshape(R_ROIS, C_CH, 7, 7)


# speedup vs baseline: 1.1093x; 1.1093x over previous
import functools

import jax
import jax.numpy as jnp
from jax import lax
from jax.experimental import pallas as pl
from jax.experimental.pallas import tpu as pltpu
from jax.experimental.pallas import tpu_sc as plsc

N_IMG, C_CH, H_F, W_F = 4, 384, 64, 64
R_ROIS = 1024
P_OUT = 49
NC, NS, L = 2, 16, 16
NW = NC * NS
RW = R_ROIS // NW
PW = RW * P_OUT
N_CHUNK = PW // L
SLAB_D = 3
SLAB_SZ = C_CH * P_OUT


def _pre_body(f_ref, t_ref):
    hw = H_F * W_F
    x = f_ref[0].reshape(C_CH, hw)
    x = x.reshape(C_CH // 32, 2, 16, hw).transpose(0, 2, 1, 3)
    t_ref[...] = x.reshape(C_CH, hw).astype(jnp.bfloat16).T


def _features_to_table(features):
    return pl.pallas_call(
        _pre_body,
        grid=(N_IMG,),
        in_specs=[pl.BlockSpec((1, C_CH, H_F, W_F), lambda n: (n, 0, 0, 0))],
        out_specs=pl.BlockSpec((H_F * W_F, C_CH), lambda n: (n, 0)),
        out_shape=jax.ShapeDtypeStruct((N_IMG * H_F * W_F, C_CH),
                                       jnp.bfloat16),
    )(features)


def _sc_body(table_hbm, roist_hbm, out_hbm,
             rois_v, pos_v, idx_v, tap_v, slab_v, sem_g0, sem_g1, sem_f):
    sem_g = (sem_g0, sem_g1)
    wid = lax.axis_index("s") * NC + lax.axis_index("c")
    i32 = jnp.int32
    f32 = jnp.float32
    iota = lax.iota(i32, L)

    pltpu.sync_copy(roist_hbm, rois_v)

    def _bvec(i):
        idx5 = (jnp.broadcast_to(i * L, (L,)).astype(i32) + iota) * 5
        return plsc.load_gather(rois_v, [idx5]).astype(i32)

    def _take16(x, idx):
        return lax.gather(
            x, idx[:, None],
            dimension_numbers=lax.GatherDimensionNumbers(
                offset_dims=(), collapsed_slice_dims=(0,),
                start_index_map=(0,)),
            slice_sizes=(1,),
            mode=lax.GatherScatterMode.PROMISE_IN_BOUNDS)

    def _csum(m):
        x = jnp.where(m, 1, 0).astype(i32)
        k = 1
        while k < L:
            shifted = _take16(x, jnp.maximum(iota - k, 0))
            x = x + jnp.where(iota >= k, shifted, 0)
            k *= 2
        return x

    def _count_body(i, t):
        b = _bvec(i)
        return tuple(t[v] + _csum(b == v)[L - 1] for v in range(N_IMG))

    zero4 = tuple(jnp.zeros((), i32) for _ in range(N_IMG))
    tot = lax.fori_loop(0, R_ROIS // L, _count_body, zero4)
    run = lax.fori_loop(0, 2 * wid, _count_body, zero4)
    off = (jnp.zeros((), i32), tot[0], tot[0] + tot[1],
           tot[0] + tot[1] + tot[2])

    for s in range(RW // L):
        b = _bvec(2 * wid + s)
        pos = jnp.zeros((L,), i32)
        new_run = []
        for v in range(N_IMG):
            m = b == v
            csum = _csum(m)
            pos = jnp.where(
                m, jnp.broadcast_to(off[v] + run[v], (L,)) + csum - 1, pos)
            new_run.append(run[v] + csum[L - 1])
        run = tuple(new_run)
        pos_v[pl.ds(s * L, L)] = pos

    def _issue(c, par):
        g0 = PW * wid + L * c
        r0 = g0 // P_OUT
        p0 = g0 - r0 * P_OUT
        pp = jnp.broadcast_to(p0, (L,)).astype(i32) + iota
        wrap = pp >= P_OUT
        r = jnp.broadcast_to(r0, (L,)).astype(i32) + jnp.where(wrap, 1, 0)
        r = jnp.minimum(r, R_ROIS - 1)
        p = pp - jnp.where(wrap, P_OUT, 0)
        r5 = r * 5
        col = lambda j: plsc.load_gather(rois_v, [r5 + j])
        x1, y1, x2, y2 = col(1), col(2), col(3), col(4)
        b = col(0).astype(i32)
        py = (p.astype(f32) * (1.0 / 7.0)).astype(i32)
        px = p - py * 7
        relx = (px.astype(f32) + 0.5) * (1.0 / 7.0)
        rely = (py.astype(f32) + 0.5) * (1.0 / 7.0)
        xs = relx * (x2 - x1) + x1
        ys = rely * (y2 - y1) + y1
        gx = xs * (0.125 * 2.0 / W_F) - 1.0
        gy = ys * (0.125 * 2.0 / H_F) - 1.0
        x = ((gx + 1.0) * W_F - 1.0) * 0.5
        y = ((gy + 1.0) * H_F - 1.0) * 0.5
        xi0 = (x + 1.0).astype(i32) - 1
        yi0 = (y + 1.0).astype(i32) - 1
        dx = x - xi0.astype(f32)
        dy = y - yi0.astype(f32)
        xi1 = xi0 + 1
        yi1 = yi0 + 1
        wx0 = jnp.where((xi0 >= 0) & (xi0 <= W_F - 1), 1.0 - dx, 0.0)
        wx1 = jnp.where(xi1 <= W_F - 1, dx, 0.0)
        wy0 = jnp.where((yi0 >= 0) & (yi0 <= H_F - 1), 1.0 - dy, 0.0)
        wy1 = jnp.where(yi1 <= H_F - 1, dy, 0.0)
        xc0 = jnp.clip(xi0, 0, W_F - 1)
        xc1 = jnp.clip(xi1, 0, W_F - 1)
        yc0 = jnp.clip(yi0, 0, H_F - 1)
        yc1 = jnp.clip(yi1, 0, H_F - 1)
        base = b * (H_F * W_F)
        idx_v[4 * par + 0, :] = base + yc0 * W_F + xc0
        idx_v[4 * par + 1, :] = base + yc1 * W_F + xc0
        idx_v[4 * par + 2, :] = base + yc0 * W_F + xc1
        idx_v[4 * par + 3, :] = base + yc1 * W_F + xc1
        for t in range(4):
            pltpu.async_copy(table_hbm.at[idx_v.at[4 * par + t]],
                             tap_v.at[4 * par + t], sem_g[par])
        return (wx0 * wy0, wx0 * wy1, wx1 * wy0, wx1 * wy1)

    def _wait_taps(par):
        for t in range(4):
            pltpu.make_async_copy(table_hbm.at[idx_v.at[4 * par + t]],
                                  tap_v.at[4 * par + t], sem_g[par]).wait()

    def _compute(c, par, weights):
        wa, wb, wc, wd = weights
        tp0 = L * c

        def _pt_body(p16, carry):
            tp = tp0 + p16
            rl = tp // P_OUT
            psc = tp - rl * P_OUT
            ysc = psc // 7
            xsc = psc - ysc * 7
            slot = rl - (rl // SLAB_D) * SLAB_D
            lane = jnp.broadcast_to(p16, (L,)).astype(i32)
            bwa = _take16(wa, lane)
            bwb = _take16(wb, lane)
            bwc = _take16(wc, lane)
            bwd = _take16(wd, lane)
            cbase = jnp.broadcast_to(slot * C_CH, (L,)).astype(i32) + iota
            pv = jnp.broadcast_to(psc, (L,)).astype(i32)
            pk = lambda w: plsc.pack(w, w, format=plsc.PackFormat.INTERLEAVED)
            pwa, pwb, pwc, pwd = pk(bwa), pk(bwb), pk(bwc), pk(bwd)
            for j in range(C_CH // (2 * L)):
                sl = pl.ds(j * 2 * L, 2 * L)
                o = (pwa * tap_v[4 * par + 0, p16, sl]
                     + pwb * tap_v[4 * par + 1, p16, sl]
                     + pwc * tap_v[4 * par + 2, p16, sl]
                     + pwd * tap_v[4 * par + 3, p16, sl])
                o0, o1 = plsc.unpack(o, format=plsc.PackFormat.INTERLEAVED)
                plsc.store_scatter(slab_v, [cbase + j * 2 * L, pv], o0)
                plsc.store_scatter(slab_v, [cbase + j * 2 * L + L, pv], o1)
            return carry

        lax.fori_loop(0, L, _pt_body, 0)

        npv = tp0 // P_OUT
        nd = (tp0 + L) // P_OUT
        @pl.when(nd > npv)
        def _flush():
            rl = nd - 1
            slot = rl - (rl // SLAB_D) * SLAB_D
            grp = rl // L
            vec = pos_v[pl.ds(grp * L, L)]
            posr = _take16(vec, jnp.broadcast_to(rl - grp * L, (L,))
                           .astype(i32))[0]
            @pl.when(npv >= 1)
            def _drain():
                pltpu.make_async_copy(
                    out_hbm.at[0], slab_v.at[pl.ds(0, C_CH)], sem_f).wait()
            pltpu.async_copy(
                slab_v.at[pl.ds(slot * C_CH, C_CH)],
                out_hbm.at[posr], sem_f)

    w0 = _issue(0, 0)

    def _pair(cc, carry):
        w_even = carry
        c0 = 2 * cc
        w_odd = _issue(c0 + 1, 1)
        _wait_taps(0)
        _compute(c0, 0, w_even)
        w_next = _issue(c0 + 2, 0)
        _wait_taps(1)
        _compute(c0 + 1, 1, w_odd)
        return w_next

    lax.fori_loop(0, N_CHUNK // 2, _pair, w0)

    pltpu.make_async_copy(out_hbm.at[0],
                          slab_v.at[pl.ds(0, C_CH)], sem_f).wait()
    _wait_taps(0)


@functools.partial(
    pl.kernel,
    out_type=jax.ShapeDtypeStruct((R_ROIS, C_CH, P_OUT), jnp.float32),
    mesh=plsc.VectorSubcoreMesh(core_axis_name="c", subcore_axis_name="s",
                                num_cores=NC, num_subcores=NS),
    compiler_params=pltpu.CompilerParams(use_tc_tiling_on_sc=False,
                                         needs_layout_passes=False),
    scratch_types=[
        pltpu.VMEM((5 * R_ROIS,), jnp.float32),
        pltpu.VMEM((RW,), jnp.int32),
        pltpu.VMEM((8, L), jnp.int32),
        pltpu.VMEM((8, L, C_CH), jnp.bfloat16),
        pltpu.VMEM((SLAB_D * C_CH, P_OUT), jnp.float32),
        pltpu.SemaphoreType.DMA,
        pltpu.SemaphoreType.DMA,
        pltpu.SemaphoreType.DMA,
    ],
)
def _sc_sample(table_hbm, roist_hbm, out_hbm, *scratch):
    _sc_body(table_hbm, roist_hbm, out_hbm, *scratch)


def kernel(features, rois):
    table = _features_to_table(features)
    out = _sc_sample(table, rois.reshape(5 * R_ROIS))
    return out.reshape(R_ROIS, C_CH, 7, 7)
